# baseline (speedup 1.0000x reference)
import jax
import jax.numpy as jnp
from jax import lax
from jax.experimental import pallas as pl
from jax.experimental.pallas import tpu as pltpu
from jax.experimental.pallas import tpu_sc as plsc

N = 10000
E = 320000
D = 128
H = 32
P = 320000

NC = 2
NS = 16
NW = NC * NS

CH = 80
EW = E // NW
WCHUNKS = EW // CH
OWN = 632
NP = NS * OWN

PW = 2 * P // NW
PCH = 128
PCHUNKS = -(-PW // PCH)
PWP = PCHUNKS * PCH

F32 = jnp.float32
I32 = jnp.int32
TCBLK = 2000


def _mm1(x, w1l_t):
    def body(x_ref, w_ref, a_ref):
        a_ref[...] = jnp.dot(x_ref[...], w_ref[...],
                             preferred_element_type=F32)

    return pl.pallas_call(
        body,
        grid=(N // TCBLK,),
        in_specs=[
            pl.BlockSpec((TCBLK, D), lambda i: (i, 0)),
            pl.BlockSpec((D, H), lambda i: (0, 0)),
        ],
        out_specs=pl.BlockSpec((TCBLK, H), lambda i: (i, 0)),
        out_shape=jax.ShapeDtypeStruct((N, H), F32),
    )(x, w1l_t)


def _combine_mm(part, cntp, x, w1r_t, b1, w2cat, b2):
    def body(p_ref, c_ref, x_ref, w1_ref, b1_ref, w2_ref, b2_ref,
             a_ref, rb_ref):
        s = p_ref[0] + p_ref[1]
        c = c_ref[0] + c_ref[1]
        rc = 1.0 / jnp.maximum(c, 1.0)
        r1 = jnp.dot(x_ref[...], w1_ref[...], preferred_element_type=F32)
        h = jnp.maximum(s * rc + b1_ref[...] + r1, 0.0)
        t = jnp.dot(h, w2_ref[...], preferred_element_type=F32)
        a_ref[...] = t[:, :H]
        rb_ref[...] = t[:, H:] + b2_ref[...]

    return pl.pallas_call(
        body,
        grid=(N // TCBLK,),
        in_specs=[
            pl.BlockSpec((NC, TCBLK, H), lambda i: (0, i, 0)),
            pl.BlockSpec((NC, TCBLK, 1), lambda i: (0, i, 0)),
            pl.BlockSpec((TCBLK, D), lambda i: (i, 0)),
            pl.BlockSpec((D, H), lambda i: (0, 0)),
            pl.BlockSpec((1, H), lambda i: (0, 0)),
            pl.BlockSpec((H, 2 * H), lambda i: (0, 0)),
            pl.BlockSpec((1, H), lambda i: (0, 0)),
        ],
        out_specs=[
            pl.BlockSpec((TCBLK, H), lambda i: (i, 0)),
            pl.BlockSpec((TCBLK, H), lambda i: (i, 0)),
        ],
        out_shape=[
            jax.ShapeDtypeStruct((N, H), F32),
            jax.ShapeDtypeStruct((NP, H), F32),
        ],
    )(part, cntp, x, w1r_t, b1, w2cat, b2)


def _seg_sum(a, edge_index, with_count):
    mesh = plsc.VectorSubcoreMesh(core_axis_name="c", subcore_axis_name="s")
    out_type = [jax.ShapeDtypeStruct((NC, NP, H), F32)]
    if with_count:
        out_type.append(jax.ShapeDtypeStruct((NC, 1, NP), F32))
    scratch = [
        pltpu.VMEM((EW,), I32),
        pltpu.VMEM((EW,), I32),
        pltpu.VMEM((2, CH, H), F32),
        pltpu.VMEM((CH,), F32),
        pltpu.VMEM((OWN, H), F32),
        pltpu.VMEM((OWN + 8,), F32),
        pltpu.VMEM_SHARED((NP, H), F32),
        pltpu.VMEM_SHARED((NP,), F32),
        pltpu.SemaphoreType.DMA,
        pltpu.SemaphoreType.DMA,
    ]

    def body(a_hbm, ei_hbm, *rest):
        if with_count:
            part_hbm, cntp_hbm = rest[0], rest[1]
            rest = rest[2:]
        else:
            part_hbm, cntp_hbm = rest[0], None
            rest = rest[1:]
        (idx_src, idx_dst, rows, ones, zb, zc, acc, acc_cnt,
         sem0, sem1) = rest

        c = lax.axis_index("c")
        s = lax.axis_index("s")
        g = c * NS + s
        off = pl.multiple_of(s * OWN, 8)

        def zrow(i, _):
            zb[i, pl.ds(0, 16)] = jnp.zeros((16,), F32)
            zb[i, pl.ds(16, 16)] = jnp.zeros((16,), F32)
            return 0
        lax.fori_loop(0, OWN, zrow, 0)
        pltpu.sync_copy(zb, acc.at[pl.ds(off, OWN)])

        if with_count:
            def zcrow(i, _):
                zc[pl.ds(i * 16, 16)] = jnp.zeros((16,), F32)
                return 0
            lax.fori_loop(0, (OWN + 8) // 16, zcrow, 0)
            pltpu.sync_copy(zc.at[pl.ds(0, OWN)], acc_cnt.at[pl.ds(off, OWN)])
            for k in range(CH // 16):
                ones[pl.ds(k * 16, 16)] = jnp.ones((16,), F32)

        plsc.subcore_barrier()

        eoff = pl.multiple_of(g * EW, 8)
        pltpu.sync_copy(ei_hbm.at[0, pl.ds(eoff, EW)], idx_src)
        pltpu.sync_copy(ei_hbm.at[1, pl.ds(eoff, EW)], idx_dst)

        cp0 = pltpu.make_async_copy(
            a_hbm.at[idx_src.at[pl.ds(0, CH)]], rows.at[0], sem0)
        cp0.start()
        cp0.wait()

        def chunk(j, _):
            slot = lax.rem(j, 2)
            nxt = lax.rem(j + 1, 2)
            coff = pl.multiple_of((j + 1) * CH, 8)
            cpn = pltpu.make_async_copy(
                a_hbm.at[idx_src.at[pl.ds(coff, CH)]], rows.at[nxt], sem1)

            @pl.when(j + 1 < WCHUNKS)
            def _():
                cpn.start()

            doff = pl.multiple_of(j * CH, 8)
            pltpu.sync_copy(rows.at[slot],
                            acc.at[idx_dst.at[pl.ds(doff, CH)]], add=True)
            if with_count:
                pltpu.sync_copy(ones,
                                acc_cnt.at[idx_dst.at[pl.ds(doff, CH)]],
                                add=True)

            @pl.when(j + 1 < WCHUNKS)
            def _():
                cpn.wait()
            return 0
        lax.fori_loop(0, WCHUNKS, chunk, 0)

        plsc.subcore_barrier()

        pltpu.sync_copy(acc.at[pl.ds(off, OWN)],
                        part_hbm.at[c, pl.ds(off, OWN)])
        if with_count:
            @pl.when(s == 0)
            def _():
                pltpu.sync_copy(acc_cnt, cntp_hbm.at[c, 0])

    fn = pl.kernel(body, out_type=out_type, mesh=mesh, scratch_types=scratch,
                   compiler_params=pltpu.CompilerParams(
                       use_tc_tiling_on_sc=False))
    return fn(a, edge_index)


def _decoder(part, cntp, rb2, pairs3):
    mesh = plsc.VectorSubcoreMesh(core_axis_name="c", subcore_axis_name="s")
    out_type = [
        jax.ShapeDtypeStruct((NW, PCHUNKS, PCH // 2), F32),
        jax.ShapeDtypeStruct((NP, H), F32),
    ]
    scratch = [
        pltpu.VMEM((PCHUNKS, PCH), I32),
        pltpu.VMEM((OWN, H), F32),
        pltpu.VMEM((OWN, H), F32),
        pltpu.VMEM((OWN, H), F32),
        pltpu.VMEM((OWN + 8,), F32),
        pltpu.VMEM((OWN + 8,), F32),
        pltpu.VMEM((PCH, H), F32),
        pltpu.VMEM((PCH, H), F32),
        pltpu.VMEM((PCHUNKS, PCH // 2), F32),
        pltpu.SemaphoreType.DMA,
        pltpu.SemaphoreType.DMA,
    ]

    def body(part_hbm, cntp_hbm, rb2_hbm, pairs_hbm, out_hbm, z_hbm,
             idx, p0, p1, zr, c0, c1, zb0, zb1, outb, sem0, sem1):
        c = lax.axis_index("c")
        s = lax.axis_index("s")
        g = c * NS + s
        off = pl.multiple_of(s * OWN, 8)

        pltpu.sync_copy(part_hbm.at[0, pl.ds(off, OWN)], p0)
        pltpu.sync_copy(part_hbm.at[1, pl.ds(off, OWN)], p1)
        pltpu.sync_copy(cntp_hbm.at[0, 0, pl.ds(off, OWN)],
                        c0.at[pl.ds(0, OWN)])
        pltpu.sync_copy(cntp_hbm.at[1, 0, pl.ds(off, OWN)],
                        c1.at[pl.ds(0, OWN)])
        pltpu.sync_copy(rb2_hbm.at[pl.ds(off, OWN)], zr)

        def zblk(base, ks, _):
            rcv = 1.0 / jnp.maximum(c0[pl.ds(base, 16)] + c1[pl.ds(base, 16)],
                                    1.0)
            for k in range(ks, 16):
                r = base + k
                rc = rcv[k]
                zr[r, pl.ds(0, 16)] = (
                    (p0[r, pl.ds(0, 16)] + p1[r, pl.ds(0, 16)]) * rc
                    + zr[r, pl.ds(0, 16)])
                zr[r, pl.ds(16, 16)] = (
                    (p0[r, pl.ds(16, 16)] + p1[r, pl.ds(16, 16)]) * rc
                    + zr[r, pl.ds(16, 16)])
            return 0
        lax.fori_loop(0, OWN // 16, lambda b, v: zblk(b * 16, 0, v), 0)
        zblk(OWN - 16, 8, 0)
        pltpu.sync_copy(zr, z_hbm.at[pl.ds(off, OWN)])

        plsc.subcore_barrier()

        pltpu.sync_copy(pairs_hbm.at[g], idx)

        def start(j, zb, sem):
            pltpu.make_async_copy(z_hbm.at[idx.at[j]], zb, sem).start()

        def drain(j, zb, sem):
            pltpu.make_async_copy(z_hbm.at[idx.at[j]], zb, sem).wait()

        bitrev = [int(f"{k:04b}"[::-1], 2) for k in range(16)]

        def compute(j, zb):
            lane = lax.iota(I32, 16)

            def pairprod(p):
                a0 = zb[2 * p, pl.ds(0, 16)]
                a1 = zb[2 * p, pl.ds(16, 16)]
                b0 = zb[2 * p + 1, pl.ds(0, 16)]
                b1 = zb[2 * p + 1, pl.ds(16, 16)]
                return a0 * b0 + a1 * b1

            for grp in range(PCH // 32):
                pbase = grp * 16
                vs = [pairprod(pbase + bitrev[k]) for k in range(16)]
                for o in (8, 4, 2, 1):
                    nv = []
                    for i in range(0, len(vs), 2):
                        ra = vs[i] + jnp.take(vs[i], lane ^ o)
                        rb = vs[i + 1] + jnp.take(vs[i + 1], lane ^ o)
                        nv.append(jnp.where((lane & o) == 0, ra, rb))
                    vs = nv
                outb[j, pl.ds(pbase, 16)] = vs[0]

        start(0, zb0, sem0)

        def pair(jj, _):
            j0 = 2 * jj
            j1 = j0 + 1
            drain(j0, zb0, sem0)

            @pl.when(j1 < PCHUNKS)
            def _():
                start(j1, zb1, sem1)

            compute(j0, zb0)

            @pl.when(j1 < PCHUNKS)
            def _():
                drain(j1, zb1, sem1)

                @pl.when(j1 + 1 < PCHUNKS)
                def _():
                    start(j1 + 1, zb0, sem0)

                compute(j1, zb1)
            return 0
        lax.fori_loop(0, (PCHUNKS + 1) // 2, pair, 0)

        pltpu.sync_copy(outb, out_hbm.at[g])

    fn = pl.kernel(body, out_type=out_type, mesh=mesh, scratch_types=scratch,
                   compiler_params=pltpu.CompilerParams(
                       use_tc_tiling_on_sc=False))
    return fn(part, cntp, rb2, pairs3)


def kernel(x, edge_index, edge_pairs, W1l, b1l, W1r, W2l, b2l, W2r):
    ei = edge_index.astype(I32)
    epf = edge_pairs.astype(I32).reshape(NW, PW)
    pad = jnp.zeros((NW, PWP - PW), I32)
    pairs3 = jnp.concatenate([epf, pad], axis=1).reshape(NW, PCHUNKS, PCH)

    w1l_t = W1l.T
    w1r_t = W1r.T
    w2cat = jnp.concatenate([W2l, W2r], axis=0).T

    a1 = _mm1(x, w1l_t)
    part1, cntp = _seg_sum(a1, ei, with_count=True)
    cntc = cntp.reshape(NC, NP, 1)[:, :N, :]
    a2, rb2 = _combine_mm(part1, cntc, x, w1r_t, b1l.reshape(1, H),
                          w2cat, b2l.reshape(1, H))
    (part2,) = _seg_sum(a2, ei, with_count=False)
    logits, _ = _decoder(part2, cntp, rb2, pairs3)
    return logits.reshape(NW, PWP // 2)[:, :PW // 2].reshape(P)


# speedup vs baseline: 1.2494x; 1.2494x over previous
import jax
import jax.numpy as jnp
from jax import lax
from jax.experimental import pallas as pl
from jax.experimental.pallas import tpu as pltpu
from jax.experimental.pallas import tpu_sc as plsc

N = 10000
E = 320000
D = 128
H = 32
P = 320000

NC = 2
NS = 16
NW = NC * NS

CH = 80
WCHUNKS = E // CH // NW
OWN = 632
NP = NS * OWN

F32 = jnp.float32
I32 = jnp.int32


def _mm_split(x, wcat, rows, blk):
    k = x.shape[1]

    def body(x_ref, w_ref, a_ref, r_ref):
        t = jnp.dot(x_ref[...], w_ref[...], preferred_element_type=F32)
        a_ref[...] = t[:, :H]
        r_ref[...] = t[:, H:]

    return pl.pallas_call(
        body,
        grid=(rows // blk,),
        in_specs=[
            pl.BlockSpec((blk, k), lambda i: (i, 0)),
            pl.BlockSpec((k, 2 * H), lambda i: (0, 0)),
        ],
        out_specs=[pl.BlockSpec((blk, H), lambda i: (i, 0))] * 2,
        out_shape=[jax.ShapeDtypeStruct((rows, H), F32)] * 2,
    )(x, wcat)


def _combine_mm(part, cntp, r1, b1, wcat, blk):
    def body(p_ref, c_ref, r_ref, b_ref, w_ref, a_ref, rr_ref):
        s = p_ref[0] + p_ref[1]
        c = c_ref[0] + c_ref[1]
        rc = 1.0 / jnp.maximum(c, 1.0)
        h = jnp.maximum(s * rc + b_ref[...] + r_ref[...], 0.0)
        t = jnp.dot(h, w_ref[...], preferred_element_type=F32)
        a_ref[...] = t[:, :H]
        rr_ref[...] = t[:, H:]

    return pl.pallas_call(
        body,
        grid=(N // blk,),
        in_specs=[
            pl.BlockSpec((NC, blk, H), lambda i: (0, i, 0)),
            pl.BlockSpec((NC, blk, 1), lambda i: (0, i, 0)),
            pl.BlockSpec((blk, H), lambda i: (i, 0)),
            pl.BlockSpec((1, H), lambda i: (0, 0)),
            pl.BlockSpec((H, 2 * H), lambda i: (0, 0)),
        ],
        out_specs=[pl.BlockSpec((blk, H), lambda i: (i, 0))] * 2,
        out_shape=[jax.ShapeDtypeStruct((N, H), F32)] * 2,
    )(part, cntp, r1, b1, wcat)


def _z_combine(part, cntp, r2, b2, blk):
    def body(p_ref, c_ref, r_ref, b_ref, z_ref):
        c = c_ref[0] + c_ref[1]
        rc = 1.0 / jnp.maximum(c, 1.0)
        z_ref[...] = (p_ref[0] + p_ref[1]) * rc + b_ref[...] + r_ref[...]

    return pl.pallas_call(
        body,
        grid=(N // blk,),
        in_specs=[
            pl.BlockSpec((NC, blk, H), lambda i: (0, i, 0)),
            pl.BlockSpec((NC, blk, 1), lambda i: (0, i, 0)),
            pl.BlockSpec((blk, H), lambda i: (i, 0)),
            pl.BlockSpec((1, H), lambda i: (0, 0)),
        ],
        out_specs=pl.BlockSpec((blk, H), lambda i: (i, 0)),
        out_shape=jax.ShapeDtypeStruct((N, H), F32),
    )(part, cntp, r2, b2)


def _seg_sum(a, src3, dst3, with_count):
    mesh = plsc.VectorSubcoreMesh(core_axis_name="c", subcore_axis_name="s")
    out_type = [jax.ShapeDtypeStruct((NC, NP, H), F32)]
    if with_count:
        out_type.append(jax.ShapeDtypeStruct((NC, 1, NP), F32))
    scratch = [
        pltpu.VMEM((WCHUNKS, CH), I32),
        pltpu.VMEM((WCHUNKS, CH), I32),
        pltpu.VMEM((2, CH, H), F32),
        pltpu.VMEM((CH,), F32),
        pltpu.VMEM((OWN, H), F32),
        pltpu.VMEM((OWN + 8, ), F32),
        pltpu.VMEM_SHARED((NP, H), F32),
        pltpu.VMEM_SHARED((NP,), F32),
        pltpu.SemaphoreType.DMA,
        pltpu.SemaphoreType.DMA,
    ]

    def body(a_hbm, src_hbm, dst_hbm, *rest):
        if with_count:
            part_hbm, cntp_hbm = rest[0], rest[1]
            rest = rest[2:]
        else:
            part_hbm, cntp_hbm = rest[0], None
            rest = rest[1:]
        (idx_src, idx_dst, rows, ones, zb, zc, acc, acc_cnt,
         sem0, sem1) = rest

        c = lax.axis_index("c")
        s = lax.axis_index("s")
        g = c * NS + s
        off = pl.multiple_of(s * OWN, 8)

        def zrow(i, _):
            zb[i, pl.ds(0, 16)] = jnp.zeros((16,), F32)
            zb[i, pl.ds(16, 16)] = jnp.zeros((16,), F32)
            return 0
        lax.fori_loop(0, OWN, zrow, 0)
        pltpu.sync_copy(zb, acc.at[pl.ds(off, OWN)])

        if with_count:
            def zcrow(i, _):
                zc[pl.ds(i * 16, 16)] = jnp.zeros((16,), F32)
                return 0
            lax.fori_loop(0, (OWN + 8) // 16, zcrow, 0)
            pltpu.sync_copy(zc.at[pl.ds(0, OWN)], acc_cnt.at[pl.ds(off, OWN)])
            for k in range(CH // 16):
                ones[pl.ds(k * 16, 16)] = jnp.ones((16,), F32)

        plsc.subcore_barrier()

        pltpu.sync_copy(src_hbm.at[g], idx_src)
        pltpu.sync_copy(dst_hbm.at[g], idx_dst)

        cp0 = pltpu.make_async_copy(a_hbm.at[idx_src.at[0]], rows.at[0], sem0)
        cp0.start()
        cp0.wait()

        def chunk(j, _):
            slot = lax.rem(j, 2)
            nxt = lax.rem(j + 1, 2)
            cpn = pltpu.make_async_copy(a_hbm.at[idx_src.at[j + 1]],
                                        rows.at[nxt], sem1)

            @pl.when(j + 1 < WCHUNKS)
            def _():
                cpn.start()

            pltpu.sync_copy(rows.at[slot], acc.at[idx_dst.at[j]], add=True)
            if with_count:
                pltpu.sync_copy(ones, acc_cnt.at[idx_dst.at[j]], add=True)

            @pl.when(j + 1 < WCHUNKS)
            def _():
                cpn.wait()
            return 0
        lax.fori_loop(0, WCHUNKS, chunk, 0)

        plsc.subcore_barrier()

        pltpu.sync_copy(acc.at[pl.ds(off, OWN)],
                        part_hbm.at[c, pl.ds(off, OWN)])
        if with_count:
            @pl.when(s == 0)
            def _():
                pltpu.sync_copy(acc_cnt, cntp_hbm.at[c, 0])

    fn = pl.kernel(body, out_type=out_type, mesh=mesh, scratch_types=scratch,
                   compiler_params=pltpu.CompilerParams(
                       use_tc_tiling_on_sc=False))
    return fn(a, src3, dst3)


def _decoder(z, ps3, pd3):
    mesh = plsc.VectorSubcoreMesh(core_axis_name="c", subcore_axis_name="s")
    scratch = [
        pltpu.VMEM((WCHUNKS, CH), I32),
        pltpu.VMEM((WCHUNKS, CH), I32),
        pltpu.VMEM((CH, H), F32),
        pltpu.VMEM((CH, H), F32),
        pltpu.VMEM((CH, H), F32),
        pltpu.VMEM((CH, H), F32),
        pltpu.VMEM((WCHUNKS, CH), F32),
        pltpu.SemaphoreType.DMA,
        pltpu.SemaphoreType.DMA,
    ]

    def body(z_hbm, ps_hbm, pd_hbm, out_hbm,
             idx_s, idx_d, zs0, zs1, zd0, zd1, outb, sem0, sem1):
        c = lax.axis_index("c")
        s = lax.axis_index("s")
        g = c * NS + s

        pltpu.sync_copy(ps_hbm.at[g], idx_s)
        pltpu.sync_copy(pd_hbm.at[g], idx_d)

        def start(j, zs, zd, sem):
            pltpu.make_async_copy(z_hbm.at[idx_s.at[j]], zs, sem).start()
            pltpu.make_async_copy(z_hbm.at[idx_d.at[j]], zd, sem).start()

        def drain(j, zs, zd, sem):
            pltpu.make_async_copy(z_hbm.at[idx_s.at[j]], zs, sem).wait()
            pltpu.make_async_copy(z_hbm.at[idx_d.at[j]], zd, sem).wait()

        bitrev = [int(f"{k:04b}"[::-1], 2) for k in range(16)]

        def compute(j, zs, zd):
            lane = lax.iota(I32, 16)

            def rowprod(r):
                a0 = zs[r, pl.ds(0, 16)]
                a1 = zs[r, pl.ds(16, 16)]
                b0 = zd[r, pl.ds(0, 16)]
                b1 = zd[r, pl.ds(16, 16)]
                return a0 * b0 + a1 * b1

            for grp in range(CH // 16):
                base = grp * 16
                vs = [rowprod(base + bitrev[k]) for k in range(16)]
                for o in (8, 4, 2, 1):
                    nv = []
                    for i in range(0, len(vs), 2):
                        ra = vs[i] + jnp.take(vs[i], lane ^ o)
                        rb = vs[i + 1] + jnp.take(vs[i + 1], lane ^ o)
                        nv.append(jnp.where((lane & o) == 0, ra, rb))
                    vs = nv
                outb[j, pl.ds(base, 16)] = vs[0]

        start(0, zs0, zd0, sem0)

        def pair(jj, _):
            j0 = 2 * jj
            j1 = j0 + 1
            drain(j0, zs0, zd0, sem0)

            @pl.when(j1 < WCHUNKS)
            def _():
                start(j1, zs1, zd1, sem1)

            compute(j0, zs0, zd0)

            @pl.when(j1 < WCHUNKS)
            def _():
                drain(j1, zs1, zd1, sem1)

                @pl.when(j1 + 1 < WCHUNKS)
                def _():
                    start(j1 + 1, zs0, zd0, sem0)

                compute(j1, zs1, zd1)
            return 0
        lax.fori_loop(0, (WCHUNKS + 1) // 2, pair, 0)

        pltpu.sync_copy(outb, out_hbm.at[g])

    fn = pl.kernel(body,
                   out_type=jax.ShapeDtypeStruct((NW, WCHUNKS, CH), F32),
                   mesh=mesh, scratch_types=scratch,
                   compiler_params=pltpu.CompilerParams(
                       use_tc_tiling_on_sc=False))
    return fn(z, ps3, pd3)


def kernel(x, edge_index, edge_pairs, W1l, b1l, W1r, W2l, b2l, W2r):
    ei = edge_index.astype(I32)
    src3 = ei[0].reshape(NW, WCHUNKS, CH)
    dst3 = ei[1].reshape(NW, WCHUNKS, CH)
    ep = edge_pairs.astype(I32)
    ps3 = ep[:, 0].reshape(NW, WCHUNKS, CH)
    pd3 = ep[:, 1].reshape(NW, WCHUNKS, CH)

    w1 = jnp.concatenate([W1l, W1r], axis=0).T
    w2 = jnp.concatenate([W2l, W2r], axis=0).T

    a1, r1 = _mm_split(x, w1, N, 400)
    part1, cntp = _seg_sum(a1, src3, dst3, with_count=True)
    cntc = cntp.reshape(NC, NP, 1)[:, :N, :]
    a2, r2 = _combine_mm(part1, cntc, r1, b1l.reshape(1, H), w2, 400)
    (part2,) = _seg_sum(a2, src3, dst3, with_count=False)
    z = _z_combine(part2, cntc, r2, b2l.reshape(1, H), 400)
    logits = _decoder(z, ps3, pd3)
    return logits.reshape(P)
